# baseline (speedup 1.0000x reference)
import functools

import jax
import jax.numpy as jnp
from jax import lax
from jax.experimental import pallas as pl
from jax.experimental.pallas import tpu as pltpu
from jax.experimental.pallas import tpu_sc as plsc

_LANES = 16
_NUM_WORKERS = 32


def _ceil_to(x: int, m: int) -> int:
    return ((x + m - 1) // m) * m


@functools.lru_cache(maxsize=None)
def _make_collate_kernel(num_mols: int, total: int):
    chunk = _ceil_to(total, _NUM_WORKERS * _LANES) // _NUM_WORKERS
    tail = total - (_NUM_WORKERS - 1) * chunk
    assert 0 < tail <= chunk and chunk % _LANES == 0 and chunk % 8 == 0
    n_pad = _ceil_to(num_mols, _LANES)
    n_vecs = n_pad // _LANES
    c_vecs = chunk // _LANES
    last = _NUM_WORKERS - 1

    mesh = plsc.VectorSubcoreMesh(core_axis_name="c", subcore_axis_name="s")

    @functools.partial(
        pl.kernel,
        mesh=mesh,
        compiler_params=pltpu.CompilerParams(
            needs_layout_passes=False, use_tc_tiling_on_sc=False
        ),
        out_type=jax.ShapeDtypeStruct((total,), jnp.int32),
        scratch_types=[
            pltpu.VMEM((n_pad,), jnp.int32),
            pltpu.VMEM((chunk,), jnp.int32),
            pltpu.SemaphoreType.DMA,
        ],
    )
    def collate_kernel(n_hbm, ids_out, n_v, marks_v, sem_n):
        wid = lax.axis_index("s") * 2 + lax.axis_index("c")
        start = wid * chunk

        n_copy = pltpu.async_copy(n_hbm, n_v, sem_n)

        zeros16 = jnp.zeros((_LANES,), jnp.int32)

        def zero_body(i, _):
            marks_v[pl.ds(i * _LANES, _LANES)] = zeros16
            return 0

        lax.fori_loop(0, c_vecs, zero_body, 0, unroll=16)
        n_copy.wait()

        lane_iota = lax.iota(jnp.int32, _LANES)
        ones16 = jnp.ones((_LANES,), jnp.int32)

        def scan_body(i, carry):
            run, base_acc = carry
            m_idx = i * _LANES + lane_iota
            v = n_v[pl.ds(i * _LANES, _LANES)]
            incl = jnp.cumsum(v) + run
            valid = m_idx < (num_mols - 1)
            j = incl - start
            in_chunk = valid & (j >= 1) & (j < chunk)
            j_safe = jnp.clip(j, 0, chunk - 1)
            plsc.store_scatter(marks_v, [j_safe], ones16, mask=in_chunk)
            base_acc = base_acc + jnp.where(valid & (incl <= start), 1, 0)
            run = run + jnp.sum(v)
            return run, base_acc

        _, base_acc = lax.fori_loop(
            0, n_vecs, scan_body, (jnp.int32(0), zeros16), unroll=16
        )
        base_id = jnp.sum(base_acc)

        def psum_body(i, carry):
            m = marks_v[pl.ds(i * _LANES, _LANES)]
            marks_v[pl.ds(i * _LANES, _LANES)] = jnp.cumsum(m) + carry
            return carry + jnp.sum(m)

        lax.fori_loop(0, c_vecs, psum_body, base_id, unroll=16)

        if tail == chunk:
            pltpu.sync_copy(marks_v, ids_out.at[pl.ds(start, chunk)])
        else:

            @pl.when(wid < last)
            def _():
                pltpu.sync_copy(marks_v, ids_out.at[pl.ds(start, chunk)])

            @pl.when(wid == last)
            def _():
                pltpu.sync_copy(
                    marks_v.at[pl.ds(0, tail)], ids_out.at[pl.ds(start, tail)]
                )

    return collate_kernel


def kernel(z, pos, e, n):
    num_mols = n.shape[0]
    total = pos.shape[0]
    n_pad = _ceil_to(num_mols, _LANES)
    collate_fn = _make_collate_kernel(num_mols, total)
    n_in = n
    if n_pad != num_mols:
        n_in = jnp.pad(n, (0, n_pad - num_mols))
    batch_ids = collate_fn(n_in)
    one = jnp.where(n[0] >= -1, jnp.float32(1.0), jnp.float32(2.0))
    one_i = jnp.where(n[0] >= -1, jnp.int32(1), jnp.int32(2))
    z_out = z * one_i
    pos_out = pos * one
    e_out = (e * one).reshape(-1, 1)
    return (z_out, pos_out, batch_ids, e_out)


# speedup vs baseline: 1.0272x; 1.0272x over previous
import functools

import jax
import jax.numpy as jnp
from jax import lax
from jax.experimental import pallas as pl
from jax.experimental.pallas import tpu as pltpu
from jax.experimental.pallas import tpu_sc as plsc

_LANES = 16
_NUM_WORKERS = 32


def _ceil_to(x: int, m: int) -> int:
    return ((x + m - 1) // m) * m


@functools.lru_cache(maxsize=None)
def _make_collate_kernel(num_mols: int, total: int):
    chunk = _ceil_to(total, _NUM_WORKERS * _LANES) // _NUM_WORKERS
    tail = total - (_NUM_WORKERS - 1) * chunk
    assert 0 < tail <= chunk and chunk % _LANES == 0 and chunk % 8 == 0
    n_pad = _ceil_to(num_mols, _LANES)
    n_vecs = n_pad // _LANES
    c_vecs = chunk // _LANES
    last = _NUM_WORKERS - 1

    mesh = plsc.VectorSubcoreMesh(core_axis_name="c", subcore_axis_name="s")

    @functools.partial(
        pl.kernel,
        mesh=mesh,
        compiler_params=pltpu.CompilerParams(
            needs_layout_passes=False, use_tc_tiling_on_sc=False
        ),
        out_type=jax.ShapeDtypeStruct((total,), jnp.int32),
        scratch_types=[
            pltpu.VMEM((n_pad,), jnp.int32),
            pltpu.VMEM((chunk,), jnp.int32),
            pltpu.SemaphoreType.DMA,
        ],
    )
    def collate_kernel(n_hbm, ids_out, n_v, marks_v, sem_n):
        wid = lax.axis_index("s") * 2 + lax.axis_index("c")
        start = wid * chunk

        n_copy = pltpu.async_copy(n_hbm, n_v, sem_n)

        zeros16 = jnp.zeros((_LANES,), jnp.int32)

        def zero_body(i, _):
            marks_v[pl.ds(i * _LANES, _LANES)] = zeros16
            return 0

        lax.fori_loop(0, c_vecs, zero_body, 0, unroll=16)
        n_copy.wait()

        lane_iota = lax.iota(jnp.int32, _LANES)
        ones16 = jnp.ones((_LANES,), jnp.int32)

        def scan_body(i, carry):
            run, base_acc = carry
            m_idx = i * _LANES + lane_iota
            v = n_v[pl.ds(i * _LANES, _LANES)]
            incl = jnp.cumsum(v) + run
            valid = m_idx < (num_mols - 1)
            j = incl - start
            in_chunk = valid & (j >= 1) & (j < chunk)
            j_safe = jnp.clip(j, 0, chunk - 1)
            plsc.store_scatter(marks_v, [j_safe], ones16, mask=in_chunk)
            base_acc = base_acc + jnp.where(valid & (incl <= start), 1, 0)
            run = run + jnp.sum(v)
            return run, base_acc

        _, base_acc = lax.fori_loop(
            0, n_vecs, scan_body, (jnp.int32(0), zeros16), unroll=8
        )
        base_id = jnp.sum(base_acc)

        def psum_body(i, carry):
            m = marks_v[pl.ds(i * _LANES, _LANES)]
            marks_v[pl.ds(i * _LANES, _LANES)] = jnp.cumsum(m) + carry
            return carry + jnp.sum(m)

        lax.fori_loop(0, c_vecs, psum_body, base_id, unroll=12)

        if tail == chunk:
            pltpu.sync_copy(marks_v, ids_out.at[pl.ds(start, chunk)])
        else:

            @pl.when(wid < last)
            def _():
                pltpu.sync_copy(marks_v, ids_out.at[pl.ds(start, chunk)])

            @pl.when(wid == last)
            def _():
                pltpu.sync_copy(
                    marks_v.at[pl.ds(0, tail)], ids_out.at[pl.ds(start, tail)]
                )

    return collate_kernel


def kernel(z, pos, e, n):
    num_mols = n.shape[0]
    total = pos.shape[0]
    n_pad = _ceil_to(num_mols, _LANES)
    collate_fn = _make_collate_kernel(num_mols, total)
    n_in = n
    if n_pad != num_mols:
        n_in = jnp.pad(n, (0, n_pad - num_mols))
    batch_ids = collate_fn(n_in)
    one = jnp.where(n[0] >= -1, jnp.float32(1.0), jnp.float32(2.0))
    one_i = jnp.where(n[0] >= -1, jnp.int32(1), jnp.int32(2))
    z_out = z * one_i
    pos_out = pos * one
    e_out = (e * one).reshape(-1, 1)
    return (z_out, pos_out, batch_ids, e_out)
